# baseline (speedup 1.0000x reference)
import jax
import jax.numpy as jnp
from jax import lax
from jax.experimental import pallas as pl
from jax.experimental.pallas import tpu as pltpu
from jax.experimental.pallas import tpu_sc as plsc

EMB = 64
T = 4
B = 4096
S = 50
ROW = (T + 1) * EMB
ROWP = 384

NC = 2
NSC = 16
NW = NC * NSC
CHUNK = 128


def _make_sc_body(nch):
    rpw = nch * CHUNK

    def _sc_body(table, idsr, xtr, out, idx_v, rows_v, xt_v, z0_v, z1_v,
                 gsem0, gsem1, xsem0, xsem1):
        wid = lax.axis_index("s") * NC + lax.axis_index("c")
        pltpu.sync_copy(idsr.at[wid], idx_v)
        pltpu.async_copy(table.at[idx_v.at[0]], rows_v.at[0], gsem0)
        pltpu.async_copy(xtr.at[wid * nch], xt_v.at[pl.ds(0, CHUNK * T)], xsem0)

        def process(k, b_):
            sem = gsem0 if b_ == 0 else gsem1
            osem = gsem1 if b_ == 0 else gsem0
            xsem = xsem0 if b_ == 0 else xsem1
            oxsem = xsem1 if b_ == 0 else xsem0

            @pl.when(k + 1 < nch)
            def _start_next():
                pltpu.async_copy(table.at[idx_v.at[k + 1]], rows_v.at[1 - b_], osem)
                pltpu.async_copy(xtr.at[wid * nch + k + 1],
                                 xt_v.at[pl.ds((1 - b_) * CHUNK * T, CHUNK * T)], oxsem)

            pltpu.make_async_copy(table.at[idx_v.at[k]], rows_v.at[b_], sem).wait()
            pltpu.make_async_copy(xtr.at[wid * nch + k],
                                  xt_v.at[pl.ds(b_ * CHUNK * T, CHUNK * T)], xsem).wait()

            z_v = z0_v if b_ == 0 else z1_v

            def row_body(i, c2, b_=b_, z_v=z_v):
                gi = jnp.broadcast_to((i * T).astype(jnp.int32), (16,))
                xb = [plsc.load_gather(xt_v, [gi + (b_ * CHUNK * T + t)])
                      for t in range(T)]
                for j in range(EMB // 16):
                    acc = rows_v[b_, i, pl.ds(T * EMB + 16 * j, 16)]
                    for t in range(T):
                        acc = acc + xb[t] * rows_v[b_, i, pl.ds(t * EMB + 16 * j, 16)]
                    z_v[pl.ds(i * EMB + 16 * j, 16)] = acc
                return c2

            lax.fori_loop(0, CHUNK, row_body, 0)
            pltpu.sync_copy(
                z_v, out.at[pl.ds((wid * rpw + k * CHUNK) * EMB, CHUNK * EMB)])

        def chunk_pair(k2, carry):
            for b_ in range(2):
                process(k2 * 2 + b_, b_)
            return carry

        lax.fori_loop(0, nch // 2, chunk_pair, 0)
        if nch % 2 == 1:
            process(jnp.int32(nch - 1), (nch - 1) % 2)

    return _sc_body


def _sc_reduce(table, idsr, xtr, nch):
    mesh = plsc.VectorSubcoreMesh(core_axis_name="c", subcore_axis_name="s")
    return pl.kernel(
        _make_sc_body(nch),
        out_type=jax.ShapeDtypeStruct((NW * nch * CHUNK * EMB,), jnp.float32),
        mesh=mesh,
        compiler_params=pltpu.CompilerParams(needs_layout_passes=False),
        scratch_types=[
            pltpu.VMEM((nch, CHUNK), jnp.int32),
            pltpu.VMEM((2, CHUNK, ROWP), jnp.float32),
            pltpu.VMEM((2 * CHUNK * T,), jnp.float32),
            pltpu.VMEM((CHUNK * EMB,), jnp.float32),
            pltpu.VMEM((CHUNK * EMB,), jnp.float32),
            pltpu.SemaphoreType.DMA,
            pltpu.SemaphoreType.DMA,
            pltpu.SemaphoreType.DMA,
            pltpu.SemaphoreType.DMA,
        ],
    )(table, idsr, xtr)


def _pad_body(t_ref, out_ref):
    out_ref[:, :ROW] = t_ref[...]
    out_ref[:, ROW:] = jnp.zeros((t_ref.shape[0], ROWP - ROW), jnp.float32)


def _pad_table(table):
    R = 1000
    return pl.pallas_call(
        _pad_body,
        grid=(table.shape[0] // R,),
        in_specs=[pl.BlockSpec((R, ROW), lambda i: (i, 0))],
        out_specs=pl.BlockSpec((R, ROWP), lambda i: (i, 0)),
        out_shape=jax.ShapeDtypeStruct((table.shape[0], ROWP), jnp.float32),
    )(table)


_INV_PI = 0.3183098861837907
_PI_HI = 3.140625
_PI_LO = 9.67653589793e-4


def _fast_sin(x):
    n = jnp.floor(x * _INV_PI + 0.5)
    y = (x - n * _PI_HI) - n * _PI_LO
    y2 = y * y
    p = y * (1.0 + y2 * (-0.16666667 + y2 * (8.3333310e-3
             + y2 * (-1.98409e-4 + y2 * 2.7526e-6))))
    m = n - 2.0 * jnp.floor(n * 0.5)
    return p * (1.0 - 2.0 * m)


BB = 64


def _tc_body(z_ref, xt_ref, w0_ref, p0_ref, out_ref):
    xtb = xt_ref[...]
    k0 = (xtb[:, :, 0:1] * w0_ref[0] + xtb[:, :, 1:2] * w0_ref[1]
          + xtb[:, :, 2:3] * w0_ref[2] + xtb[:, :, 3:4] * w0_ref[3]
          + p0_ref[0])
    out_ref[:, :, 0:1] = k0
    out_ref[:, :, 1:] = _fast_sin(z_ref[...])


def _tc_body_alias(z_ref, xt_ref, w0_ref, p0_ref, prev_ref, out_ref):
    del prev_ref
    _tc_body(z_ref, xt_ref, w0_ref, p0_ref, out_ref)


def _tc_finish_half(z2, x_time, w0f, p0, off, prev=None):
    nb = z2.shape[0]
    in_specs = [
        pl.BlockSpec((BB, S, EMB), lambda i: (i, 0, 0)),
        pl.BlockSpec((BB, S, T), lambda i, off=off: (i + off, 0, 0)),
        pl.BlockSpec(memory_space=pltpu.SMEM),
        pl.BlockSpec(memory_space=pltpu.SMEM),
    ]
    args = [z2, x_time, w0f, p0]
    body = _tc_body
    kwargs = {}
    if prev is not None:
        in_specs.append(pl.BlockSpec(memory_space=pl.ANY))
        args.append(prev)
        body = _tc_body_alias
        kwargs["input_output_aliases"] = {4: 0}
    return pl.pallas_call(
        body,
        grid=(nb // BB,),
        in_specs=in_specs,
        out_specs=pl.BlockSpec((BB, S, 1 + EMB), lambda i, off=off: (i + off, 0, 0)),
        out_shape=jax.ShapeDtypeStruct((B, S, 1 + EMB), jnp.float32),
        **kwargs,
    )(*args)


HB = B // 2
NCHH = HB * S // NW // CHUNK


def kernel(x_ser, x_time, table, W_omega0, W_phi0):
    table_p = lax.dynamic_update_slice(
        jnp.zeros((table.shape[0], ROWP), jnp.float32), table, (0, 0))
    w0f = W_omega0.reshape(T)
    ids0 = x_ser[:HB].reshape(NW, NCHH, CHUNK).astype(jnp.int32)
    ids1 = x_ser[HB:].reshape(NW, NCHH, CHUNK).astype(jnp.int32)
    xt0 = x_time[:HB].reshape(NW * NCHH, CHUNK * T)
    xt1 = x_time[HB:].reshape(NW * NCHH, CHUNK * T)
    z0 = _sc_reduce(table_p, ids0, xt0, NCHH).reshape(HB, S, EMB)
    z1 = _sc_reduce(table_p, ids1, xt1, NCHH).reshape(HB, S, EMB)
    out = _tc_finish_half(z0, x_time, w0f, W_phi0, 0)
    out = _tc_finish_half(z1, x_time, w0f, W_phi0, HB // BB, prev=out)
    return out


# speedup vs baseline: 1.2938x; 1.2938x over previous
import jax
import jax.numpy as jnp
from jax import lax
from jax.experimental import pallas as pl
from jax.experimental.pallas import tpu as pltpu
from jax.experimental.pallas import tpu_sc as plsc

EMB = 64
T = 4
B = 4096
S = 50
ROW = (T + 1) * EMB
ROWP = 384

NC = 2
NSC = 16
NW = NC * NSC
CHUNK = 128


def _make_sc_body(nch):
    rpw = nch * CHUNK

    def _sc_body(table, idsr, xtr, out, idx_v, rows_v, xt_v, z0_v, z1_v,
                 gsem0, gsem1, xsem0, xsem1):
        wid = lax.axis_index("s") * NC + lax.axis_index("c")
        pltpu.sync_copy(idsr.at[wid], idx_v)
        pltpu.async_copy(table.at[idx_v.at[0]], rows_v.at[0], gsem0)
        pltpu.async_copy(xtr.at[wid * nch], xt_v.at[pl.ds(0, CHUNK * T)], xsem0)

        def process(k, b_):
            sem = gsem0 if b_ == 0 else gsem1
            osem = gsem1 if b_ == 0 else gsem0
            xsem = xsem0 if b_ == 0 else xsem1
            oxsem = xsem1 if b_ == 0 else xsem0

            @pl.when(k + 1 < nch)
            def _start_next():
                pltpu.async_copy(table.at[idx_v.at[k + 1]], rows_v.at[1 - b_], osem)
                pltpu.async_copy(xtr.at[wid * nch + k + 1],
                                 xt_v.at[pl.ds((1 - b_) * CHUNK * T, CHUNK * T)], oxsem)

            pltpu.make_async_copy(table.at[idx_v.at[k]], rows_v.at[b_], sem).wait()
            pltpu.make_async_copy(xtr.at[wid * nch + k],
                                  xt_v.at[pl.ds(b_ * CHUNK * T, CHUNK * T)], xsem).wait()

            z_v = z0_v if b_ == 0 else z1_v

            def row_body(i, c2, b_=b_, z_v=z_v):
                gi = jnp.broadcast_to((i * T).astype(jnp.int32), (16,))
                xb = [plsc.load_gather(xt_v, [gi + (b_ * CHUNK * T + t)])
                      for t in range(T)]
                for j in range(EMB // 16):
                    acc = rows_v[b_, i, pl.ds(T * EMB + 16 * j, 16)]
                    for t in range(T):
                        acc = acc + xb[t] * rows_v[b_, i, pl.ds(t * EMB + 16 * j, 16)]
                    z_v[pl.ds(i * EMB + 16 * j, 16)] = acc
                return c2

            lax.fori_loop(0, CHUNK, row_body, 0)
            pltpu.sync_copy(
                z_v, out.at[pl.ds((wid * rpw + k * CHUNK) * EMB, CHUNK * EMB)])

        def chunk_pair(k2, carry):
            for b_ in range(2):
                process(k2 * 2 + b_, b_)
            return carry

        lax.fori_loop(0, nch // 2, chunk_pair, 0)
        if nch % 2 == 1:
            process(jnp.int32(nch - 1), (nch - 1) % 2)

    return _sc_body


def _sc_reduce(table, idsr, xtr, nch):
    mesh = plsc.VectorSubcoreMesh(core_axis_name="c", subcore_axis_name="s")
    return pl.kernel(
        _make_sc_body(nch),
        out_type=jax.ShapeDtypeStruct((NW * nch * CHUNK * EMB,), jnp.float32),
        mesh=mesh,
        compiler_params=pltpu.CompilerParams(needs_layout_passes=False),
        scratch_types=[
            pltpu.VMEM((nch, CHUNK), jnp.int32),
            pltpu.VMEM((2, CHUNK, ROWP), jnp.float32),
            pltpu.VMEM((2 * CHUNK * T,), jnp.float32),
            pltpu.VMEM((CHUNK * EMB,), jnp.float32),
            pltpu.VMEM((CHUNK * EMB,), jnp.float32),
            pltpu.SemaphoreType.DMA,
            pltpu.SemaphoreType.DMA,
            pltpu.SemaphoreType.DMA,
            pltpu.SemaphoreType.DMA,
        ],
    )(table, idsr, xtr)


def _pad_body(t_ref, out_ref):
    out_ref[:, :ROW] = t_ref[...]
    out_ref[:, ROW:] = jnp.zeros((t_ref.shape[0], ROWP - ROW), jnp.float32)


def _pad_table(table):
    R = 1000
    return pl.pallas_call(
        _pad_body,
        grid=(table.shape[0] // R,),
        in_specs=[pl.BlockSpec((R, ROW), lambda i: (i, 0))],
        out_specs=pl.BlockSpec((R, ROWP), lambda i: (i, 0)),
        out_shape=jax.ShapeDtypeStruct((table.shape[0], ROWP), jnp.float32),
    )(table)


_INV_PI = 0.3183098861837907
_PI_HI = 3.140625
_PI_LO = 9.67653589793e-4


def _fast_sin(x):
    n = jnp.floor(x * _INV_PI + 0.5)
    y = (x - n * _PI_HI) - n * _PI_LO
    y2 = y * y
    p = y * (1.0 + y2 * (-0.16666667 + y2 * (8.3333310e-3
             + y2 * (-1.98409e-4 + y2 * 2.7526e-6))))
    m = n - 2.0 * jnp.floor(n * 0.5)
    return p * (1.0 - 2.0 * m)


BB = 32


def _tc_body(z_ref, xt_ref, w0_ref, p0_ref, out_ref):
    xtb = xt_ref[...]
    k0 = (xtb[:, :, 0:1] * w0_ref[0] + xtb[:, :, 1:2] * w0_ref[1]
          + xtb[:, :, 2:3] * w0_ref[2] + xtb[:, :, 3:4] * w0_ref[3]
          + p0_ref[0])
    out_ref[:, :, 0:1] = k0
    out_ref[:, :, 1:] = _fast_sin(z_ref[...])


def _tc_body_alias(z_ref, xt_ref, w0_ref, p0_ref, prev_ref, out_ref):
    del prev_ref
    _tc_body(z_ref, xt_ref, w0_ref, p0_ref, out_ref)


def _tc_finish_half(z2, x_time, w0f, p0, off, prev=None):
    nb = z2.shape[0]
    in_specs = [
        pl.BlockSpec((BB, S, EMB), lambda i: (i, 0, 0)),
        pl.BlockSpec((BB, S, T), lambda i, off=off: (i + off, 0, 0)),
        pl.BlockSpec(memory_space=pltpu.SMEM),
        pl.BlockSpec(memory_space=pltpu.SMEM),
    ]
    args = [z2, x_time, w0f, p0]
    body = _tc_body
    kwargs = {}
    if prev is not None:
        in_specs.append(pl.BlockSpec(memory_space=pl.ANY))
        args.append(prev)
        body = _tc_body_alias
        kwargs["input_output_aliases"] = {4: 0}
    return pl.pallas_call(
        body,
        grid=(nb // BB,),
        in_specs=in_specs,
        out_specs=pl.BlockSpec((BB, S, 1 + EMB), lambda i, off=off: (i + off, 0, 0)),
        out_shape=jax.ShapeDtypeStruct((B, S, 1 + EMB), jnp.float32),
        **kwargs,
    )(*args)


HB = B // 2
NCHH = HB * S // NW // CHUNK


def kernel(x_ser, x_time, table, W_omega0, W_phi0):
    table_p = _pad_table(table)
    w0f = W_omega0.reshape(T)
    ids0 = x_ser[:HB].reshape(NW, NCHH, CHUNK).astype(jnp.int32)
    ids1 = x_ser[HB:].reshape(NW, NCHH, CHUNK).astype(jnp.int32)
    xt0 = x_time[:HB].reshape(NW * NCHH, CHUNK * T)
    xt1 = x_time[HB:].reshape(NW * NCHH, CHUNK * T)
    z0 = _sc_reduce(table_p, ids0, xt0, NCHH).reshape(HB, S, EMB)
    z1 = _sc_reduce(table_p, ids1, xt1, NCHH).reshape(HB, S, EMB)
    out = _tc_finish_half(z0, x_time, w0f, W_phi0, 0)
    out = _tc_finish_half(z1, x_time, w0f, W_phi0, HB // BB, prev=out)
    return out


# speedup vs baseline: 1.3382x; 1.0343x over previous
import jax
import jax.numpy as jnp
from jax import lax
from jax.experimental import pallas as pl
from jax.experimental.pallas import tpu as pltpu
from jax.experimental.pallas import tpu_sc as plsc

EMB = 64
T = 4
B = 4096
S = 50
ROW = (T + 1) * EMB
ROWP = 384

NC = 2
NSC = 16
NW = NC * NSC
CHUNK = 128


def _make_sc_body(nch):
    rpw = nch * CHUNK

    def _sc_body(table, idsr, xtr, out, idx_v, rows_v, xt_v, z0_v, z1_v,
                 gsem0, gsem1, xsem0, xsem1):
        wid = lax.axis_index("s") * NC + lax.axis_index("c")
        pltpu.sync_copy(idsr.at[wid], idx_v)
        pltpu.async_copy(table.at[idx_v.at[0]], rows_v.at[0], gsem0)
        pltpu.async_copy(xtr.at[wid * nch], xt_v.at[pl.ds(0, CHUNK * T)], xsem0)

        def process(k, b_):
            sem = gsem0 if b_ == 0 else gsem1
            osem = gsem1 if b_ == 0 else gsem0
            xsem = xsem0 if b_ == 0 else xsem1
            oxsem = xsem1 if b_ == 0 else xsem0

            @pl.when(k + 1 < nch)
            def _start_next():
                pltpu.async_copy(table.at[idx_v.at[k + 1]], rows_v.at[1 - b_], osem)
                pltpu.async_copy(xtr.at[wid * nch + k + 1],
                                 xt_v.at[pl.ds((1 - b_) * CHUNK * T, CHUNK * T)], oxsem)

            pltpu.make_async_copy(table.at[idx_v.at[k]], rows_v.at[b_], sem).wait()
            pltpu.make_async_copy(xtr.at[wid * nch + k],
                                  xt_v.at[pl.ds(b_ * CHUNK * T, CHUNK * T)], xsem).wait()

            z_v = z0_v if b_ == 0 else z1_v

            def row_body(i, c2, b_=b_, z_v=z_v):
                gi = jnp.broadcast_to((i * T).astype(jnp.int32), (16,))
                xb = [plsc.load_gather(xt_v, [gi + (b_ * CHUNK * T + t)])
                      for t in range(T)]
                for j in range(EMB // 16):
                    acc = rows_v[b_, i, pl.ds(T * EMB + 16 * j, 16)]
                    for t in range(T):
                        acc = acc + xb[t] * rows_v[b_, i, pl.ds(t * EMB + 16 * j, 16)]
                    z_v[pl.ds(i * EMB + 16 * j, 16)] = acc
                return c2

            lax.fori_loop(0, CHUNK, row_body, 0)
            pltpu.sync_copy(
                z_v, out.at[pl.ds((wid * rpw + k * CHUNK) * EMB, CHUNK * EMB)])

        def chunk_pair(k2, carry):
            for b_ in range(2):
                process(k2 * 2 + b_, b_)
            return carry

        lax.fori_loop(0, nch // 2, chunk_pair, 0)
        if nch % 2 == 1:
            process(jnp.int32(nch - 1), (nch - 1) % 2)

    return _sc_body


def _sc_reduce(table, idsr, xtr, nch):
    mesh = plsc.VectorSubcoreMesh(core_axis_name="c", subcore_axis_name="s")
    return pl.kernel(
        _make_sc_body(nch),
        out_type=jax.ShapeDtypeStruct((NW * nch * CHUNK * EMB,), jnp.float32),
        mesh=mesh,
        compiler_params=pltpu.CompilerParams(needs_layout_passes=False),
        scratch_types=[
            pltpu.VMEM((nch, CHUNK), jnp.int32),
            pltpu.VMEM((2, CHUNK, ROWP), jnp.float32),
            pltpu.VMEM((2 * CHUNK * T,), jnp.float32),
            pltpu.VMEM((CHUNK * EMB,), jnp.float32),
            pltpu.VMEM((CHUNK * EMB,), jnp.float32),
            pltpu.SemaphoreType.DMA,
            pltpu.SemaphoreType.DMA,
            pltpu.SemaphoreType.DMA,
            pltpu.SemaphoreType.DMA,
        ],
    )(table, idsr, xtr)


def _pad_body(t_ref, out_ref):
    out_ref[:, :ROW] = t_ref[...]
    out_ref[:, ROW:] = jnp.zeros((t_ref.shape[0], ROWP - ROW), jnp.float32)


def _pad_table(table):
    R = 1000
    return pl.pallas_call(
        _pad_body,
        grid=(table.shape[0] // R,),
        in_specs=[pl.BlockSpec((R, ROW), lambda i: (i, 0))],
        out_specs=pl.BlockSpec((R, ROWP), lambda i: (i, 0)),
        out_shape=jax.ShapeDtypeStruct((table.shape[0], ROWP), jnp.float32),
    )(table)


_INV_PI = 0.3183098861837907
_PI_HI = 3.140625
_PI_LO = 9.67653589793e-4


def _fast_sin(x):
    n = jnp.floor(x * _INV_PI + 0.5)
    y = (x - n * _PI_HI) - n * _PI_LO
    y2 = y * y
    p = y * (1.0 + y2 * (-0.16666667 + y2 * (8.3333310e-3
             + y2 * (-1.98409e-4 + y2 * 2.7526e-6))))
    m = n - 2.0 * jnp.floor(n * 0.5)
    return p * (1.0 - 2.0 * m)


BB = 64


def _tc_body(z_ref, xt_ref, w0_ref, p0_ref, out_ref):
    xtb = xt_ref[...]
    k0 = (xtb[:, :, 0:1] * w0_ref[0] + xtb[:, :, 1:2] * w0_ref[1]
          + xtb[:, :, 2:3] * w0_ref[2] + xtb[:, :, 3:4] * w0_ref[3]
          + p0_ref[0])
    out_ref[:, :, 0:1] = k0
    out_ref[:, :, 1:] = _fast_sin(z_ref[...])


def _tc_body_alias(z_ref, xt_ref, w0_ref, p0_ref, prev_ref, out_ref):
    del prev_ref
    _tc_body(z_ref, xt_ref, w0_ref, p0_ref, out_ref)


def _tc_finish_half(z2, x_time, w0f, p0, off, prev=None):
    nb = z2.shape[0]
    in_specs = [
        pl.BlockSpec((BB, S, EMB), lambda i: (i, 0, 0)),
        pl.BlockSpec((BB, S, T), lambda i, off=off: (i + off, 0, 0)),
        pl.BlockSpec(memory_space=pltpu.SMEM),
        pl.BlockSpec(memory_space=pltpu.SMEM),
    ]
    args = [z2, x_time, w0f, p0]
    body = _tc_body
    kwargs = {}
    if prev is not None:
        in_specs.append(pl.BlockSpec(memory_space=pl.ANY))
        args.append(prev)
        body = _tc_body_alias
        kwargs["input_output_aliases"] = {4: 0}
    return pl.pallas_call(
        body,
        grid=(nb // BB,),
        in_specs=in_specs,
        out_specs=pl.BlockSpec((BB, S, 1 + EMB), lambda i, off=off: (i + off, 0, 0)),
        out_shape=jax.ShapeDtypeStruct((B, S, 1 + EMB), jnp.float32),
        **kwargs,
    )(*args)


HB = B // 2
NCHH = HB * S // NW // CHUNK


def kernel(x_ser, x_time, table, W_omega0, W_phi0):
    table_p = _pad_table(table)
    w0f = W_omega0.reshape(T)
    ids0 = x_ser[:HB].reshape(NW, NCHH, CHUNK).astype(jnp.int32)
    ids1 = x_ser[HB:].reshape(NW, NCHH, CHUNK).astype(jnp.int32)
    xt0 = x_time[:HB].reshape(NW * NCHH, CHUNK * T)
    xt1 = x_time[HB:].reshape(NW * NCHH, CHUNK * T)
    z0 = _sc_reduce(table_p, ids0, xt0, NCHH).reshape(HB, S, EMB)
    z1 = _sc_reduce(table_p, ids1, xt1, NCHH).reshape(HB, S, EMB)
    out = _tc_finish_half(z0, x_time, w0f, W_phi0, 0)
    out = _tc_finish_half(z1, x_time, w0f, W_phi0, HB // BB, prev=out)
    return out


# speedup vs baseline: 1.3466x; 1.0063x over previous
import jax
import jax.numpy as jnp
from jax import lax
from jax.experimental import pallas as pl
from jax.experimental.pallas import tpu as pltpu
from jax.experimental.pallas import tpu_sc as plsc

EMB = 64
T = 4
B = 4096
S = 50
ROW = (T + 1) * EMB
ROWP = 384

NC = 2
NSC = 16
NW = NC * NSC
CHUNK = 128


def _make_sc_body(nch):
    rpw = nch * CHUNK

    def _sc_body(table, idsr, xtr, out, idx_v, rows_v, xt_v, z0_v, z1_v,
                 gsem0, gsem1, xsem0, xsem1):
        wid = lax.axis_index("s") * NC + lax.axis_index("c")
        pltpu.sync_copy(idsr.at[wid], idx_v)
        pltpu.async_copy(table.at[idx_v.at[0]], rows_v.at[0], gsem0)
        pltpu.async_copy(xtr.at[wid * nch], xt_v.at[pl.ds(0, CHUNK * T)], xsem0)

        def process(k, b_):
            sem = gsem0 if b_ == 0 else gsem1
            osem = gsem1 if b_ == 0 else gsem0
            xsem = xsem0 if b_ == 0 else xsem1
            oxsem = xsem1 if b_ == 0 else xsem0

            @pl.when(k + 1 < nch)
            def _start_next():
                pltpu.async_copy(table.at[idx_v.at[k + 1]], rows_v.at[1 - b_], osem)
                pltpu.async_copy(xtr.at[wid * nch + k + 1],
                                 xt_v.at[pl.ds((1 - b_) * CHUNK * T, CHUNK * T)], oxsem)

            pltpu.make_async_copy(table.at[idx_v.at[k]], rows_v.at[b_], sem).wait()
            pltpu.make_async_copy(xtr.at[wid * nch + k],
                                  xt_v.at[pl.ds(b_ * CHUNK * T, CHUNK * T)], xsem).wait()

            z_v = z0_v if b_ == 0 else z1_v

            def row_body(i, c2, b_=b_, z_v=z_v):
                gi = jnp.broadcast_to((i * T).astype(jnp.int32), (16,))
                xb = [plsc.load_gather(xt_v, [gi + (b_ * CHUNK * T + t)])
                      for t in range(T)]
                for j in range(EMB // 16):
                    acc = rows_v[b_, i, pl.ds(T * EMB + 16 * j, 16)]
                    for t in range(T):
                        acc = acc + xb[t] * rows_v[b_, i, pl.ds(t * EMB + 16 * j, 16)]
                    z_v[pl.ds(i * EMB + 16 * j, 16)] = acc
                return c2

            lax.fori_loop(0, CHUNK, row_body, 0)
            pltpu.sync_copy(
                z_v, out.at[pl.ds((wid * rpw + k * CHUNK) * EMB, CHUNK * EMB)])

        def chunk_pair(k2, carry):
            for b_ in range(2):
                process(k2 * 2 + b_, b_)
            return carry

        lax.fori_loop(0, nch // 2, chunk_pair, 0)
        if nch % 2 == 1:
            process(jnp.int32(nch - 1), (nch - 1) % 2)

    return _sc_body


def _sc_reduce(table, idsr, xtr, nch):
    mesh = plsc.VectorSubcoreMesh(core_axis_name="c", subcore_axis_name="s")
    return pl.kernel(
        _make_sc_body(nch),
        out_type=jax.ShapeDtypeStruct((NW * nch * CHUNK * EMB,), jnp.float32),
        mesh=mesh,
        compiler_params=pltpu.CompilerParams(needs_layout_passes=False),
        scratch_types=[
            pltpu.VMEM((nch, CHUNK), jnp.int32),
            pltpu.VMEM((2, CHUNK, ROWP), jnp.float32),
            pltpu.VMEM((2 * CHUNK * T,), jnp.float32),
            pltpu.VMEM((CHUNK * EMB,), jnp.float32),
            pltpu.VMEM((CHUNK * EMB,), jnp.float32),
            pltpu.SemaphoreType.DMA,
            pltpu.SemaphoreType.DMA,
            pltpu.SemaphoreType.DMA,
            pltpu.SemaphoreType.DMA,
        ],
    )(table, idsr, xtr)


def _pad_body(t_ref, out_ref):
    out_ref[:, :ROW] = t_ref[...]
    out_ref[:, ROW:] = jnp.zeros((t_ref.shape[0], ROWP - ROW), jnp.float32)


def _pad_table(table):
    R = 1000
    return pl.pallas_call(
        _pad_body,
        grid=(table.shape[0] // R,),
        in_specs=[pl.BlockSpec((R, ROW), lambda i: (i, 0))],
        out_specs=pl.BlockSpec((R, ROWP), lambda i: (i, 0)),
        out_shape=jax.ShapeDtypeStruct((table.shape[0], ROWP), jnp.float32),
    )(table)


_INV_PI = 0.3183098861837907
_PI_HI = 3.140625
_PI_LO = 9.67653589793e-4


def _fast_sin(x):
    n = jnp.floor(x * _INV_PI + 0.5)
    y = (x - n * _PI_HI) - n * _PI_LO
    y2 = y * y
    p = y * (1.0 + y2 * (-0.16666667 + y2 * (8.3333310e-3
             + y2 * (-1.98409e-4 + y2 * 2.7526e-6))))
    m = n - 2.0 * jnp.floor(n * 0.5)
    return p * (1.0 - 2.0 * m)


BB = 128


def _tc_body(z_ref, xt_ref, w0_ref, p0_ref, out_ref):
    xtb = xt_ref[...]
    k0 = (xtb[:, :, 0:1] * w0_ref[0] + xtb[:, :, 1:2] * w0_ref[1]
          + xtb[:, :, 2:3] * w0_ref[2] + xtb[:, :, 3:4] * w0_ref[3]
          + p0_ref[0])
    out_ref[:, :, 0:1] = k0
    out_ref[:, :, 1:] = _fast_sin(z_ref[...])


def _tc_body_alias(z_ref, xt_ref, w0_ref, p0_ref, prev_ref, out_ref):
    del prev_ref
    _tc_body(z_ref, xt_ref, w0_ref, p0_ref, out_ref)


def _tc_finish_half(z2, x_time, w0f, p0, off, prev=None):
    nb = z2.shape[0]
    in_specs = [
        pl.BlockSpec((BB, S, EMB), lambda i: (i, 0, 0)),
        pl.BlockSpec((BB, S, T), lambda i, off=off: (i + off, 0, 0)),
        pl.BlockSpec(memory_space=pltpu.SMEM),
        pl.BlockSpec(memory_space=pltpu.SMEM),
    ]
    args = [z2, x_time, w0f, p0]
    body = _tc_body
    kwargs = {}
    if prev is not None:
        in_specs.append(pl.BlockSpec(memory_space=pl.ANY))
        args.append(prev)
        body = _tc_body_alias
        kwargs["input_output_aliases"] = {4: 0}
    return pl.pallas_call(
        body,
        grid=(nb // BB,),
        in_specs=in_specs,
        out_specs=pl.BlockSpec((BB, S, 1 + EMB), lambda i, off=off: (i + off, 0, 0)),
        out_shape=jax.ShapeDtypeStruct((B, S, 1 + EMB), jnp.float32),
        **kwargs,
    )(*args)


HB = B // 2
NCHH = HB * S // NW // CHUNK


def kernel(x_ser, x_time, table, W_omega0, W_phi0):
    table_p = _pad_table(table)
    w0f = W_omega0.reshape(T)
    ids0 = x_ser[:HB].reshape(NW, NCHH, CHUNK).astype(jnp.int32)
    ids1 = x_ser[HB:].reshape(NW, NCHH, CHUNK).astype(jnp.int32)
    xt0 = x_time[:HB].reshape(NW * NCHH, CHUNK * T)
    xt1 = x_time[HB:].reshape(NW * NCHH, CHUNK * T)
    z0 = _sc_reduce(table_p, ids0, xt0, NCHH).reshape(HB, S, EMB)
    z1 = _sc_reduce(table_p, ids1, xt1, NCHH).reshape(HB, S, EMB)
    out = _tc_finish_half(z0, x_time, w0f, W_phi0, 0)
    out = _tc_finish_half(z1, x_time, w0f, W_phi0, HB // BB, prev=out)
    return out


# speedup vs baseline: 1.3467x; 1.0000x over previous
import jax
import jax.numpy as jnp
from jax import lax
from jax.experimental import pallas as pl
from jax.experimental.pallas import tpu as pltpu
from jax.experimental.pallas import tpu_sc as plsc

EMB = 64
T = 4
B = 4096
S = 50
ROW = (T + 1) * EMB
ROWP = 384

NC = 2
NSC = 16
NW = NC * NSC
CHUNK = 128


def _make_sc_body(nch):
    rpw = nch * CHUNK

    def _sc_body(table, idsr, xtr, out, idx_v, rows_v, xt_v, z0_v, z1_v,
                 gsem0, gsem1, xsem0, xsem1):
        wid = lax.axis_index("s") * NC + lax.axis_index("c")
        pltpu.sync_copy(idsr.at[wid], idx_v)
        pltpu.async_copy(table.at[idx_v.at[0]], rows_v.at[0], gsem0)
        pltpu.async_copy(xtr.at[wid * nch], xt_v.at[pl.ds(0, CHUNK * T)], xsem0)

        def process(k, b_):
            sem = gsem0 if b_ == 0 else gsem1
            osem = gsem1 if b_ == 0 else gsem0
            xsem = xsem0 if b_ == 0 else xsem1
            oxsem = xsem1 if b_ == 0 else xsem0

            @pl.when(k + 1 < nch)
            def _start_next():
                pltpu.async_copy(table.at[idx_v.at[k + 1]], rows_v.at[1 - b_], osem)
                pltpu.async_copy(xtr.at[wid * nch + k + 1],
                                 xt_v.at[pl.ds((1 - b_) * CHUNK * T, CHUNK * T)], oxsem)

            pltpu.make_async_copy(table.at[idx_v.at[k]], rows_v.at[b_], sem).wait()
            pltpu.make_async_copy(xtr.at[wid * nch + k],
                                  xt_v.at[pl.ds(b_ * CHUNK * T, CHUNK * T)], xsem).wait()

            z_v = z0_v if b_ == 0 else z1_v

            def row_body(i, c2, b_=b_, z_v=z_v):
                gi = jnp.broadcast_to((i * T).astype(jnp.int32), (16,))
                xb = [plsc.load_gather(xt_v, [gi + (b_ * CHUNK * T + t)])
                      for t in range(T)]
                for j in range(EMB // 16):
                    acc = rows_v[b_, i, pl.ds(T * EMB + 16 * j, 16)]
                    for t in range(T):
                        acc = acc + xb[t] * rows_v[b_, i, pl.ds(t * EMB + 16 * j, 16)]
                    z_v[pl.ds(i * EMB + 16 * j, 16)] = acc
                return c2

            lax.fori_loop(0, CHUNK, row_body, 0)
            pltpu.sync_copy(
                z_v, out.at[pl.ds((wid * rpw + k * CHUNK) * EMB, CHUNK * EMB)])

        def chunk_pair(k2, carry):
            for b_ in range(2):
                process(k2 * 2 + b_, b_)
            return carry

        lax.fori_loop(0, nch // 2, chunk_pair, 0)
        if nch % 2 == 1:
            process(jnp.int32(nch - 1), (nch - 1) % 2)

    return _sc_body


def _sc_reduce(table, idsr, xtr, nch):
    mesh = plsc.VectorSubcoreMesh(core_axis_name="c", subcore_axis_name="s")
    return pl.kernel(
        _make_sc_body(nch),
        out_type=jax.ShapeDtypeStruct((NW * nch * CHUNK * EMB,), jnp.float32),
        mesh=mesh,
        compiler_params=pltpu.CompilerParams(needs_layout_passes=False),
        scratch_types=[
            pltpu.VMEM((nch, CHUNK), jnp.int32),
            pltpu.VMEM((2, CHUNK, ROWP), jnp.float32),
            pltpu.VMEM((2 * CHUNK * T,), jnp.float32),
            pltpu.VMEM((CHUNK * EMB,), jnp.float32),
            pltpu.VMEM((CHUNK * EMB,), jnp.float32),
            pltpu.SemaphoreType.DMA,
            pltpu.SemaphoreType.DMA,
            pltpu.SemaphoreType.DMA,
            pltpu.SemaphoreType.DMA,
        ],
    )(table, idsr, xtr)


def _pad_body(t_ref, out_ref):
    out_ref[:, :ROW] = t_ref[...]
    if ROWP > ROW:
        out_ref[:, ROW:] = jnp.zeros((t_ref.shape[0], ROWP - ROW), jnp.float32)


def _pad_table(table):
    R = 1000
    return pl.pallas_call(
        _pad_body,
        grid=(table.shape[0] // R,),
        in_specs=[pl.BlockSpec((R, ROW), lambda i: (i, 0))],
        out_specs=pl.BlockSpec((R, ROWP), lambda i: (i, 0)),
        out_shape=jax.ShapeDtypeStruct((table.shape[0], ROWP), jnp.float32),
    )(table)


_INV_PI = 0.3183098861837907
_PI_HI = 3.140625
_PI_LO = 9.67653589793e-4


def _fast_sin(x):
    n = jnp.floor(x * _INV_PI + 0.5)
    y = (x - n * _PI_HI) - n * _PI_LO
    y2 = y * y
    p = y * (1.0 + y2 * (-0.16666667 + y2 * (8.3333310e-3
             + y2 * (-1.98409e-4 + y2 * 2.7526e-6))))
    m = n - 2.0 * jnp.floor(n * 0.5)
    return p * (1.0 - 2.0 * m)


BB = 128


def _tc_body(z_ref, xt_ref, w0_ref, p0_ref, out_ref):
    xtb = xt_ref[...]
    k0 = (xtb[:, :, 0:1] * w0_ref[0] + xtb[:, :, 1:2] * w0_ref[1]
          + xtb[:, :, 2:3] * w0_ref[2] + xtb[:, :, 3:4] * w0_ref[3]
          + p0_ref[0])
    out_ref[:, :, 0:1] = k0
    out_ref[:, :, 1:] = _fast_sin(z_ref[...])


def _tc_body_alias(z_ref, xt_ref, w0_ref, p0_ref, prev_ref, out_ref):
    del prev_ref
    _tc_body(z_ref, xt_ref, w0_ref, p0_ref, out_ref)


def _tc_finish_half(z2, x_time, w0f, p0, off, prev=None):
    nb = z2.shape[0]
    in_specs = [
        pl.BlockSpec((BB, S, EMB), lambda i: (i, 0, 0)),
        pl.BlockSpec((BB, S, T), lambda i, off=off: (i + off, 0, 0)),
        pl.BlockSpec(memory_space=pltpu.SMEM),
        pl.BlockSpec(memory_space=pltpu.SMEM),
    ]
    args = [z2, x_time, w0f, p0]
    body = _tc_body
    kwargs = {}
    if prev is not None:
        in_specs.append(pl.BlockSpec(memory_space=pl.ANY))
        args.append(prev)
        body = _tc_body_alias
        kwargs["input_output_aliases"] = {4: 0}
    return pl.pallas_call(
        body,
        grid=(nb // BB,),
        in_specs=in_specs,
        out_specs=pl.BlockSpec((BB, S, 1 + EMB), lambda i, off=off: (i + off, 0, 0)),
        out_shape=jax.ShapeDtypeStruct((B, S, 1 + EMB), jnp.float32),
        **kwargs,
    )(*args)


HB = B // 2
NCHH = HB * S // NW // CHUNK


def kernel(x_ser, x_time, table, W_omega0, W_phi0):
    table_p = _pad_table(table)
    w0f = W_omega0.reshape(T)
    ids0 = x_ser[:HB].reshape(NW, NCHH, CHUNK).astype(jnp.int32)
    ids1 = x_ser[HB:].reshape(NW, NCHH, CHUNK).astype(jnp.int32)
    xt0 = x_time[:HB].reshape(NW * NCHH, CHUNK * T)
    xt1 = x_time[HB:].reshape(NW * NCHH, CHUNK * T)
    z0 = _sc_reduce(table_p, ids0, xt0, NCHH).reshape(HB, S, EMB)
    z1 = _sc_reduce(table_p, ids1, xt1, NCHH).reshape(HB, S, EMB)
    out = _tc_finish_half(z0, x_time, w0f, W_phi0, 0)
    out = _tc_finish_half(z1, x_time, w0f, W_phi0, HB // BB, prev=out)
    return out
